# baseline (speedup 1.0000x reference)
import functools

import jax
import jax.numpy as jnp
from jax import lax
from jax.experimental import pallas as pl
from jax.experimental.pallas import tpu as pltpu
from jax.experimental.pallas import tpu_sc as plsc

C = 96
IH = IW = 224
NPIX = IH * IW
OH = OW = 448
ONPIX = OH * OW
K = 4
LANES = 16

NSUB = 16
NCORE = 2
PIX_PER_TILE = NPIX // NSUB
CG = 8
NGRP = C // CG
GRP_PER_CORE = NGRP // NCORE
CHUNK = 56
NCHUNK = PIX_PER_TILE // CHUNK
NPAIR = NCHUNK // 2
ROWS = CHUNK * LANES
WB = ONPIX // NSUB
TCH = 784


def _sc_body(x_hbm, smp_hbm, zero_hbm, out_hbm,
             smp_c, xc, rows, idxb, tin, tout, acc,
             sem_smp, sem_x, sem_sc, sem_wb, sem_z):
    cid = lax.axis_index("c")
    sid = lax.axis_index("s")
    p0 = sid * PIX_PER_TILE

    lane = lax.iota(jnp.int32, LANES)
    cb_x = lane & 1
    cb_y = (lane >> 1) & 1
    mx = cb_x == 1
    my = cb_y == 1
    samp_off = (lane >> 2) * 2

    for g in range(GRP_PER_CORE):
        gg = cid * GRP_PER_CORE + g
        ch0 = gg * CG

        def start_in(b, ci):
            base = p0 + ci * CHUNK
            pltpu.async_copy(smp_hbm.at[pl.ds(base, CHUNK), :],
                             smp_c.at[b], sem_smp.at[b])
            pltpu.async_copy(x_hbm.at[pl.ds(ch0, CG), pl.ds(base, CHUNK)],
                             xc.at[b], sem_x.at[b])

        start_in(0, 0)
        start_in(1, 1)
        if g == 0:
            pltpu.sync_copy(zero_hbm, acc.at[pl.ds(sid * WB, WB)])
        plsc.subcore_barrier()

        def pair_body(ci2, _):
            for b in range(2):
                ci = ci2 * 2 + b

                @pl.when(ci2 > 0)
                def _():
                    pltpu.make_async_copy(
                        rows.at[b], acc.at[idxb.at[b]], sem_sc.at[b]).wait()

                pltpu.make_async_copy(smp_hbm.at[pl.ds(0, CHUNK), :],
                                      smp_c.at[b], sem_smp.at[b]).wait()
                pltpu.make_async_copy(x_hbm.at[pl.ds(0, CG), pl.ds(0, CHUNK)],
                                      xc.at[b], sem_x.at[b]).wait()

                def pix_body(jl, _):
                    jv = jnp.full((LANES,), jl, jnp.int32)
                    sx = plsc.load_gather(smp_c.at[b], [jv, samp_off])
                    sy = plsc.load_gather(smp_c.at[b], [jv, samp_off + 1])
                    x0i = sx.astype(jnp.int32)
                    y0i = sy.astype(jnp.int32)
                    fx = sx - x0i.astype(jnp.float32)
                    fy = sy - y0i.astype(jnp.float32)
                    wx = jnp.where(mx, fx, 1.0 - fx)
                    wy = jnp.where(my, fy, 1.0 - fy)
                    w = wx * wy * (1.0 / K)
                    oidx = (y0i + cb_y) * OW + (x0i + cb_x)
                    jl16 = jl * LANES
                    idxb.at[b][pl.ds(jl16, LANES)] = oidx
                    rvec = jl16 + lane
                    for cc in range(CG):
                        cv = jnp.full((LANES,), cc, jnp.int32)
                        xcv = plsc.load_gather(xc.at[b], [cv, jv])
                        plsc.store_scatter(rows.at[b], [rvec, cv], w * xcv)
                    return 0

                lax.fori_loop(0, CHUNK, pix_body, 0)
                pltpu.async_copy(rows.at[b], acc.at[idxb.at[b]],
                                 sem_sc.at[b], add=True)

                @pl.when(ci2 < NPAIR - 1)
                def _():
                    start_in(b, ci + 2)
            return 0

        lax.fori_loop(0, NPAIR, pair_body, 0)
        for b in range(2):
            pltpu.make_async_copy(
                rows.at[b], acc.at[idxb.at[b]], sem_sc.at[b]).wait()
        plsc.subcore_barrier()

        def wb_body(tc, _):
            col0 = sid * WB + tc * TCH
            pltpu.sync_copy(acc.at[pl.ds(col0, TCH)], tin)

            def tr_body(jl, _):
                pv = jl * LANES + lane
                for cc in range(CG):
                    cv = jnp.full((LANES,), cc, jnp.int32)
                    v = plsc.load_gather(tin, [pv, cv])
                    tout.at[cc][pl.ds(jl * LANES, LANES)] = v
                return 0

            lax.fori_loop(0, TCH // LANES, tr_body, 0)
            if g != GRP_PER_CORE - 1:
                pltpu.async_copy(zero_hbm.at[pl.ds(tc * TCH, TCH)],
                                 acc.at[pl.ds(col0, TCH)], sem_z)
            for cc in range(CG):
                pltpu.async_copy(tout.at[cc],
                                 out_hbm.at[ch0 + cc, pl.ds(col0, TCH)],
                                 sem_wb)
            for cc in range(CG):
                pltpu.make_async_copy(
                    tout.at[cc], out_hbm.at[ch0 + cc, pl.ds(col0, TCH)],
                    sem_wb).wait()
            if g != GRP_PER_CORE - 1:
                pltpu.make_async_copy(zero_hbm.at[pl.ds(tc * TCH, TCH)],
                                      acc.at[pl.ds(col0, TCH)], sem_z).wait()
            return 0

        lax.fori_loop(0, WB // TCH, wb_body, 0)
        plsc.subcore_barrier()


@jax.jit
def _sc_unpool(x2d, smp, zero):
    mesh = plsc.VectorSubcoreMesh(core_axis_name="c", subcore_axis_name="s")
    f = pl.kernel(
        _sc_body,
        out_type=jax.ShapeDtypeStruct((C, ONPIX), jnp.float32),
        mesh=mesh,
        compiler_params=pltpu.CompilerParams(use_tc_tiling_on_sc=False,
                                             needs_layout_passes=False),
        scratch_types=[
            pltpu.VMEM((2, CHUNK, 2 * K), jnp.float32),
            pltpu.VMEM((2, CG, CHUNK), jnp.float32),
            pltpu.VMEM((2, ROWS, CG), jnp.float32),
            pltpu.VMEM((2, ROWS), jnp.int32),
            pltpu.VMEM((TCH, CG), jnp.float32),
            pltpu.VMEM((CG, TCH), jnp.float32),
            pltpu.VMEM_SHARED((ONPIX, CG), jnp.float32),
            pltpu.SemaphoreType.DMA((2,)),
            pltpu.SemaphoreType.DMA((2,)),
            pltpu.SemaphoreType.DMA((2,)),
            pltpu.SemaphoreType.DMA,
            pltpu.SemaphoreType.DMA,
        ],
    )
    return f(x2d, smp, zero)


def kernel(x, oh, ow, sample_map):
    x2d = x.reshape(C, NPIX)
    smp = sample_map.reshape(NPIX, 2 * K)
    zero = jnp.zeros((WB, CG), jnp.float32)
    buf = _sc_unpool(x2d, smp, zero)
    return buf.reshape(1, C, OH, OW)


# speedup vs baseline: 2.7494x; 2.7494x over previous
import functools

import jax
import jax.numpy as jnp
from jax import lax
from jax.experimental import pallas as pl
from jax.experimental.pallas import tpu as pltpu
from jax.experimental.pallas import tpu_sc as plsc

C = 96
IH = IW = 224
NPIX = IH * IW
OH = OW = 448
ONPIX = OH * OW
K = 4
LANES = 16

NSUB = 16
NCORE = 2
PIX_PER_TILE = NPIX // NSUB
CG = 8
NGRP = C // CG
GRP_PER_CORE = NGRP // NCORE
CHUNK = 56
NCHUNK = PIX_PER_TILE // CHUNK
NPAIR = NCHUNK // 2
ROWS = CHUNK * LANES
WB = ONPIX // NSUB
TCH = 784


def _sc_body(x_hbm, smp_hbm, zero_hbm, out_hbm,
             smp_c, xc, rows, idxb, tin, tout, acc,
             sem_smp, sem_x, sem_sc, sem_wb):
    cid = lax.axis_index("c")
    sid = lax.axis_index("s")
    p0 = sid * PIX_PER_TILE

    lane = lax.iota(jnp.int32, LANES)
    cb_x = lane & 1
    cb_y = (lane >> 1) & 1
    mx = cb_x == 1
    my = cb_y == 1
    samp_off = (lane >> 2) * 2

    for g in range(GRP_PER_CORE):
        gg = cid * GRP_PER_CORE + g
        ch0 = gg * CG

        def start_in(b, ci):
            base = p0 + ci * CHUNK
            pltpu.async_copy(smp_hbm.at[pl.ds(base, CHUNK), :],
                             smp_c.at[b], sem_smp.at[b])
            pltpu.async_copy(x_hbm.at[pl.ds(ch0, CG), pl.ds(base, CHUNK)],
                             xc.at[b], sem_x.at[b])

        start_in(0, 0)
        start_in(1, 1)
        pltpu.sync_copy(zero_hbm, acc.at[pl.ds(sid * WB, WB)])
        plsc.subcore_barrier()

        def pair_body(ci2, _):
            for b in range(2):
                ci = ci2 * 2 + b

                @pl.when(ci2 > 0)
                def _():
                    pltpu.make_async_copy(
                        rows.at[b], acc.at[idxb.at[b]], sem_sc.at[b]).wait()

                pltpu.make_async_copy(smp_hbm.at[pl.ds(0, CHUNK), :],
                                      smp_c.at[b], sem_smp.at[b]).wait()
                pltpu.make_async_copy(x_hbm.at[pl.ds(0, CG), pl.ds(0, CHUNK)],
                                      xc.at[b], sem_x.at[b]).wait()

                @plsc.parallel_loop(0, CHUNK, unroll=2)
                def pix_body(jl):
                    jv = jnp.full((LANES,), jl, jnp.int32)
                    sx = plsc.load_gather(smp_c.at[b], [jv, samp_off])
                    sy = plsc.load_gather(smp_c.at[b], [jv, samp_off + 1])
                    x0i = sx.astype(jnp.int32)
                    y0i = sy.astype(jnp.int32)
                    fx = sx - x0i.astype(jnp.float32)
                    fy = sy - y0i.astype(jnp.float32)
                    wx = jnp.where(mx, fx, 1.0 - fx)
                    wy = jnp.where(my, fy, 1.0 - fy)
                    w = wx * wy * (1.0 / K)
                    oidx = (y0i + cb_y) * OW + (x0i + cb_x)
                    jl16 = jl * LANES
                    idxb.at[b][pl.ds(jl16, LANES)] = oidx
                    rvec = jl16 + lane
                    for cc in range(CG):
                        cv = jnp.full((LANES,), cc, jnp.int32)
                        xcv = plsc.load_gather(xc.at[b], [cv, jv])
                        plsc.store_scatter(rows.at[b], [rvec, cv], w * xcv)

                pltpu.async_copy(rows.at[b], acc.at[idxb.at[b]],
                                 sem_sc.at[b], add=True)

                @pl.when(ci2 < NPAIR - 1)
                def _():
                    start_in(b, ci + 2)
            return 0

        lax.fori_loop(0, NPAIR, pair_body, 0)
        for b in range(2):
            pltpu.make_async_copy(
                rows.at[b], acc.at[idxb.at[b]], sem_sc.at[b]).wait()
        plsc.subcore_barrier()

        def wb_body(tc, _):
            col0 = sid * WB + tc * TCH
            pltpu.sync_copy(acc.at[pl.ds(col0, TCH)], tin)

            @plsc.parallel_loop(0, TCH // LANES, unroll=2)
            def tr_body(jl):
                pv = jl * LANES + lane
                for cc in range(CG):
                    cv = jnp.full((LANES,), cc, jnp.int32)
                    v = plsc.load_gather(tin, [pv, cv])
                    tout.at[cc][pl.ds(jl * LANES, LANES)] = v

            for cc in range(CG):
                pltpu.async_copy(tout.at[cc],
                                 out_hbm.at[ch0 + cc, pl.ds(col0, TCH)],
                                 sem_wb)
            for cc in range(CG):
                pltpu.make_async_copy(
                    tout.at[cc], out_hbm.at[ch0 + cc, pl.ds(col0, TCH)],
                    sem_wb).wait()
            return 0

        lax.fori_loop(0, WB // TCH, wb_body, 0)
        plsc.subcore_barrier()


@jax.jit
def _sc_unpool(x2d, smp, zero):
    mesh = plsc.VectorSubcoreMesh(core_axis_name="c", subcore_axis_name="s")
    f = pl.kernel(
        _sc_body,
        out_type=jax.ShapeDtypeStruct((C, ONPIX), jnp.float32),
        mesh=mesh,
        compiler_params=pltpu.CompilerParams(use_tc_tiling_on_sc=False,
                                             needs_layout_passes=False),
        scratch_types=[
            pltpu.VMEM((2, CHUNK, 2 * K), jnp.float32),
            pltpu.VMEM((2, CG, CHUNK), jnp.float32),
            pltpu.VMEM((2, ROWS, CG), jnp.float32),
            pltpu.VMEM((2, ROWS), jnp.int32),
            pltpu.VMEM((TCH, CG), jnp.float32),
            pltpu.VMEM((CG, TCH), jnp.float32),
            pltpu.VMEM_SHARED((ONPIX, CG), jnp.float32),
            pltpu.SemaphoreType.DMA((2,)),
            pltpu.SemaphoreType.DMA((2,)),
            pltpu.SemaphoreType.DMA((2,)),
            pltpu.SemaphoreType.DMA,
        ],
    )
    return f(x2d, smp, zero)


def kernel(x, oh, ow, sample_map):
    x2d = x.reshape(C, NPIX)
    smp = sample_map.reshape(NPIX, 2 * K)
    zero = jnp.zeros((WB, CG), jnp.float32)
    buf = _sc_unpool(x2d, smp, zero)
    return buf.reshape(1, C, OH, OW)
